# baseline (speedup 1.0000x reference)
import functools

import jax
import jax.numpy as jnp
from jax import lax
from jax.experimental import pallas as pl
from jax.experimental.pallas import tpu as pltpu
from jax.experimental.pallas import tpu_sc as plsc

NC = 2
NS = 16
EPC = 128
SEG = 40
SPLIT = (80, 80)
LANES = 128
_SC_PARAMS = None


def _sc_params():
    global _SC_PARAMS
    if _SC_PARAMS is None:
        _SC_PARAMS = pltpu.CompilerParams(needs_layout_passes=False)
    return _SC_PARAMS


def _pad_sizes(n, e):
    npad = ((n + 1 + LANES - 1) // LANES) * LANES
    npad = ((npad + NS * LANES - 1) // (NS * LANES)) * (NS * LANES)
    total = (e + EPC - 1) // EPC
    unit = NS * (SPLIT[0] + SPLIT[1])
    scale = (total + unit - 1) // unit
    na, nb = SPLIT[0] * scale, SPLIT[1] * scale
    return npad, na, nb


@functools.lru_cache(maxsize=None)
def _deg_call(npad, na, nb):
    nr = npad // LANES
    rpt = nr // NS
    pool = NS * (na + nb)
    mesh = plsc.VectorSubcoreMesh(core_axis_name="c", subcore_axis_name="s")

    def body(dst_hbm, deg_out, dstv, hist, rowidx, sdeg):
        cid = lax.axis_index("c")
        sid = lax.axis_index("s")
        my_n = jnp.where(cid == 0, na, nb)
        my_base = pl.multiple_of(
            jnp.where(cid == 0, sid * na, NS * na + sid * nb), 8)
        z16 = jnp.zeros((16,), jnp.float32)

        def zero_row(r, carry):
            for c in range(LANES // 16):
                hist[r, pl.ds(c * 16, 16)] = z16
            return carry

        lax.fori_loop(0, nr, zero_row, 0)
        for c in range(nr // 16):
            rowidx[pl.ds(c * 16, 16)] = lax.iota(jnp.int32, 16) + c * 16

        pltpu.sync_copy(hist.at[pl.ds(sid * rpt, rpt)],
                        sdeg.at[pl.ds(sid * rpt, rpt)])
        plsc.subcore_barrier()

        ones = jnp.ones((16,), jnp.float32)

        def seg_body(s, carry):
            off = pl.multiple_of(my_base + s * SEG, 8)
            pltpu.sync_copy(dst_hbm.at[pl.ds(off, SEG)], dstv)

            def acc(i, c2):
                r = i // (EPC // 16)
                c = i % (EPC // 16)
                idx = dstv[r, pl.ds(c * 16, 16)]
                row = jnp.right_shift(idx, 7)
                col = jnp.bitwise_and(idx, LANES - 1)
                plsc.addupdate_scatter(hist, [row, col], ones)
                return c2

            lax.fori_loop(0, SEG * (EPC // 16), acc, 0)
            return carry

        lax.fori_loop(0, my_n // SEG, seg_body, 0)

        pltpu.sync_copy(hist, sdeg.at[rowidx], add=True)
        plsc.subcore_barrier()

        @pl.when(sid == 0)
        def _():
            pltpu.sync_copy(sdeg, deg_out.at[cid])

    return pl.kernel(
        body,
        mesh=mesh,
        out_type=jax.ShapeDtypeStruct((NC, nr, LANES), jnp.float32),
        scratch_types=[
            pltpu.VMEM((SEG, EPC), jnp.int32),
            pltpu.VMEM((nr, LANES), jnp.float32),
            pltpu.VMEM((nr,), jnp.int32),
            pltpu.VMEM_SHARED((nr, LANES), jnp.float32),
        ],
        compiler_params=_sc_params(),
    )


@functools.lru_cache(maxsize=None)
def _scatter_call(npad, na, nb, h):
    rpt = npad // NS
    mesh = plsc.VectorSubcoreMesh(core_axis_name="c", subcore_axis_name="s")

    def body(g_hbm, src_hbm, dst_hbm, s_out,
             srcv, dstv, rows0, rows1, s_sh, sem0, sem1, sem2, sem3):
        cid = lax.axis_index("c")
        sid = lax.axis_index("s")
        my_n = jnp.where(cid == 0, na, nb)
        my_base = pl.multiple_of(
            jnp.where(cid == 0, sid * na, NS * na + sid * nb), 8)

        z16 = jnp.zeros((16,), jnp.float32)

        def zero_row(r, carry):
            for c in range(h // 16):
                rows0[r, pl.ds(c * 16, 16)] = z16
            return carry

        lax.fori_loop(0, EPC, zero_row, 0)

        def zs(k, carry):
            pltpu.sync_copy(rows0, s_sh.at[pl.ds(sid * rpt + k * EPC, EPC)])
            return carry

        lax.fori_loop(0, rpt // EPC, zs, 0)
        plsc.subcore_barrier()

        def seg_body(s, carry):
            off = pl.multiple_of(my_base + s * SEG, 8)
            pltpu.sync_copy(src_hbm.at[pl.ds(off, SEG)], srcv)
            pltpu.sync_copy(dst_hbm.at[pl.ds(off, SEG)], dstv)
            pltpu.async_copy(g_hbm.at[srcv.at[0]], rows0, sem0)
            pltpu.async_copy(g_hbm.at[srcv.at[1]], rows1, sem1)

            def step(t, c2):
                j = 2 * t
                pltpu.make_async_copy(g_hbm.at[srcv.at[0]], rows0, sem0).wait()
                pltpu.async_copy(rows0, s_sh.at[dstv.at[j]], sem2, add=True)
                pltpu.make_async_copy(g_hbm.at[srcv.at[1]], rows1, sem1).wait()
                pltpu.async_copy(rows1, s_sh.at[dstv.at[j + 1]], sem3,
                                 add=True)
                pltpu.make_async_copy(rows0, s_sh.at[dstv.at[0]], sem2).wait()

                @pl.when(j + 2 < SEG)
                def _():
                    pltpu.async_copy(g_hbm.at[srcv.at[j + 2]], rows0, sem0)

                pltpu.make_async_copy(rows1, s_sh.at[dstv.at[1]], sem3).wait()

                @pl.when(j + 3 < SEG)
                def _():
                    pltpu.async_copy(g_hbm.at[srcv.at[j + 3]], rows1, sem1)

                return c2

            lax.fori_loop(0, SEG // 2, step, 0)
            return carry

        lax.fori_loop(0, my_n // SEG, seg_body, 0)
        plsc.subcore_barrier()
        pltpu.sync_copy(s_sh.at[pl.ds(sid * rpt, rpt)],
                        s_out.at[cid, pl.ds(sid * rpt, rpt)])

    return pl.kernel(
        body,
        mesh=mesh,
        out_type=jax.ShapeDtypeStruct((NC, npad, h), jnp.float32),
        scratch_types=[
            pltpu.VMEM((SEG, EPC), jnp.int32),
            pltpu.VMEM((SEG, EPC), jnp.int32),
            pltpu.VMEM((EPC, h), jnp.float32),
            pltpu.VMEM((EPC, h), jnp.float32),
            pltpu.VMEM_SHARED((npad, h), jnp.float32),
            pltpu.SemaphoreType.DMA,
            pltpu.SemaphoreType.DMA,
            pltpu.SemaphoreType.DMA,
            pltpu.SemaphoreType.DMA,
        ],
        compiler_params=_sc_params(),
    )


BLK = 512


def _t1_body(x_ref, w_ref, da_ref, db_ref, o_ref):
    dinv = lax.rsqrt(da_ref[...] + db_ref[...] + 1.0)
    o_ref[...] = jnp.dot(x_ref[...], w_ref[...],
                         preferred_element_type=jnp.float32) * dinv


def _t2_body(n, sa_ref, sb_ref, g_ref, da_ref, db_ref, b_ref, w_ref, o_ref):
    i = pl.program_id(0)
    dinv = lax.rsqrt(da_ref[...] + db_ref[...] + 1.0)
    h1 = jnp.tanh((sa_ref[0] + sb_ref[0] + g_ref[...]) * dinv + b_ref[...])
    g2 = jnp.dot(h1, w_ref[...], preferred_element_type=jnp.float32) * dinv
    rows = lax.broadcasted_iota(jnp.int32, (BLK, 1), 0) + i * BLK
    o_ref[...] = jnp.where(rows < n, g2, 0.0)


def _t3_body(sa_ref, sb_ref, g_ref, da_ref, db_ref, b_ref, w_ref, bf_ref,
             o_ref):
    dinv = lax.rsqrt(da_ref[...] + db_ref[...] + 1.0)
    h2 = jnp.tanh((sa_ref[0] + sb_ref[0] + g_ref[...]) * dinv + b_ref[...])
    o_ref[...] = jnp.tanh(jnp.dot(h2, w_ref[...],
                                  preferred_element_type=jnp.float32)
                          + bf_ref[...])


def _row_spec(i_dim):
    return pl.BlockSpec((BLK, i_dim), lambda i: (i, 0))


def _part_spec(c):
    return pl.BlockSpec((1, BLK, H_BLK), lambda i, c=c: (c, i, 0))


H_BLK = 128


def _full_spec(a, b):
    return pl.BlockSpec((a, b), lambda i: (0, 0))


def kernel(x, edge_index, W1, b1, W2, b2, Wfc, bfc):
    n, d = x.shape
    h = W1.shape[1]
    e = edge_index.shape[1]
    npad, na, nb = _pad_sizes(n, e)
    pool = NS * (na + nb)
    epad = pool * EPC

    src = edge_index[0].astype(jnp.int32)
    dst = edge_index[1].astype(jnp.int32)
    pad_cycle = n + (jnp.arange(epad - e, dtype=jnp.int32) % (npad - n))
    pad_src = pad_cycle
    pad_dst = pad_cycle
    src_p = jnp.concatenate([src, pad_src]).reshape(pool, EPC)
    dst_p = jnp.concatenate([dst, pad_dst]).reshape(pool, EPC)
    x_p = jnp.pad(x, ((0, npad - n), (0, 0)))

    deg2 = _deg_call(npad, na, nb)(dst_p)
    dega = deg2[0].reshape(npad, 1)
    degb = deg2[1].reshape(npad, 1)

    grid = (npad // BLK,)

    g1 = pl.pallas_call(
        _t1_body,
        grid=grid,
        in_specs=[_row_spec(d), _full_spec(d, h), _row_spec(1), _row_spec(1)],
        out_specs=_row_spec(h),
        out_shape=jax.ShapeDtypeStruct((npad, h), jnp.float32),
    )(x_p, W1, dega, degb)

    scat = _scatter_call(npad, na, nb, h)
    s1 = scat(g1, src_p, dst_p)

    g2 = pl.pallas_call(
        functools.partial(_t2_body, n),
        grid=grid,
        in_specs=[_part_spec(0), _part_spec(1), _row_spec(h), _row_spec(1),
                  _row_spec(1), _full_spec(1, h), _full_spec(h, h)],
        out_specs=_row_spec(h),
        out_shape=jax.ShapeDtypeStruct((npad, h), jnp.float32),
    )(s1, s1, g1, dega, degb, b1.reshape(1, h), W2)

    s2 = scat(g2, src_p, dst_p)

    out = pl.pallas_call(
        _t3_body,
        grid=grid,
        in_specs=[_part_spec(0), _part_spec(1), _row_spec(h), _row_spec(1),
                  _row_spec(1), _full_spec(1, h), _full_spec(h, h),
                  _full_spec(1, h)],
        out_specs=_row_spec(h),
        out_shape=jax.ShapeDtypeStruct((npad, h), jnp.float32),
    )(s2, s2, g2, dega, degb, b2.reshape(1, h), Wfc,
      bfc.reshape(1, h))

    return out[:n]


# speedup vs baseline: 1.2125x; 1.2125x over previous
import functools

import jax
import jax.numpy as jnp
from jax import lax
from jax.experimental import pallas as pl
from jax.experimental.pallas import tpu as pltpu
from jax.experimental.pallas import tpu_sc as plsc

NC = 2
NS = 16
EPC = 128
SEG = 40
SPLIT = (80, 80)
LANES = 128
_SC_PARAMS = None


def _sc_params():
    global _SC_PARAMS
    if _SC_PARAMS is None:
        _SC_PARAMS = pltpu.CompilerParams(needs_layout_passes=False)
    return _SC_PARAMS


def _pad_sizes(n, e):
    npad = ((n + 1 + LANES - 1) // LANES) * LANES
    npad = ((npad + NS * LANES - 1) // (NS * LANES)) * (NS * LANES)
    total = (e + EPC - 1) // EPC
    unit = NS * (SPLIT[0] + SPLIT[1])
    scale = (total + unit - 1) // unit
    na, nb = SPLIT[0] * scale, SPLIT[1] * scale
    return npad, na, nb


@functools.lru_cache(maxsize=None)
def _deg_call(npad, na, nb):
    nr = npad // LANES
    rpt = nr // NS
    pool = NS * (na + nb)
    mesh = plsc.VectorSubcoreMesh(core_axis_name="c", subcore_axis_name="s")

    def body(dst_hbm, deg_out, dstv, hist, rowidx, sdeg):
        cid = lax.axis_index("c")
        sid = lax.axis_index("s")
        my_n = jnp.where(cid == 0, na, nb)
        my_base = pl.multiple_of(
            jnp.where(cid == 0, sid * na, NS * na + sid * nb), 8)
        z16 = jnp.zeros((16,), jnp.float32)

        def zero_row(r, carry):
            for c in range(LANES // 16):
                hist[r, pl.ds(c * 16, 16)] = z16
            return carry

        lax.fori_loop(0, nr, zero_row, 0)
        for c in range(nr // 16):
            rowidx[pl.ds(c * 16, 16)] = lax.iota(jnp.int32, 16) + c * 16

        pltpu.sync_copy(hist.at[pl.ds(sid * rpt, rpt)],
                        sdeg.at[pl.ds(sid * rpt, rpt)])
        plsc.subcore_barrier()

        ones = jnp.ones((16,), jnp.float32)

        def seg_body(s, carry):
            off = pl.multiple_of(my_base + s * SEG, 8)
            pltpu.sync_copy(dst_hbm.at[pl.ds(off, SEG)], dstv)

            def acc(i, c2):
                r = i // (EPC // 16)
                c = i % (EPC // 16)
                idx = dstv[r, pl.ds(c * 16, 16)]
                row = jnp.right_shift(idx, 7)
                col = jnp.bitwise_and(idx, LANES - 1)
                plsc.addupdate_scatter(hist, [row, col], ones)
                return c2

            lax.fori_loop(0, SEG * (EPC // 16), acc, 0)
            return carry

        lax.fori_loop(0, my_n // SEG, seg_body, 0)

        pltpu.sync_copy(hist, sdeg.at[rowidx], add=True)
        plsc.subcore_barrier()

        @pl.when(sid == 0)
        def _():
            pltpu.sync_copy(sdeg, deg_out.at[cid])

    return pl.kernel(
        body,
        mesh=mesh,
        out_type=jax.ShapeDtypeStruct((NC, nr, LANES), jnp.float32),
        scratch_types=[
            pltpu.VMEM((SEG, EPC), jnp.int32),
            pltpu.VMEM((nr, LANES), jnp.float32),
            pltpu.VMEM((nr,), jnp.int32),
            pltpu.VMEM_SHARED((nr, LANES), jnp.float32),
        ],
        compiler_params=_sc_params(),
    )


@functools.lru_cache(maxsize=None)
def _scatter_call(npad, na, nb, h):
    rpt = npad // NS
    mesh = plsc.VectorSubcoreMesh(core_axis_name="c", subcore_axis_name="s")

    def body(g_hbm, src_hbm, dst_hbm, s_out,
             srcv, dstv, rows0, rows1, s_sh, sem0, sem1, sem2, sem3):
        cid = lax.axis_index("c")
        sid = lax.axis_index("s")
        my_n = jnp.where(cid == 0, na, nb)
        my_base = pl.multiple_of(
            jnp.where(cid == 0, sid * na, NS * na + sid * nb), 8)

        z16 = jnp.zeros((16,), jnp.float32)

        def zero_row(r, carry):
            for c in range(h // 16):
                rows0[r, pl.ds(c * 16, 16)] = z16
            return carry

        lax.fori_loop(0, EPC, zero_row, 0)

        def zs(k, carry):
            pltpu.sync_copy(rows0, s_sh.at[pl.ds(sid * rpt + k * EPC, EPC)])
            return carry

        lax.fori_loop(0, rpt // EPC, zs, 0)
        plsc.subcore_barrier()

        hc = EPC // 2

        def gather2(j, buf, sa, sb):
            pltpu.async_copy(g_hbm.at[srcv.at[j, pl.ds(0, hc)]],
                             buf.at[pl.ds(0, hc)], sa)
            pltpu.async_copy(g_hbm.at[srcv.at[j, pl.ds(hc, hc)]],
                             buf.at[pl.ds(hc, hc)], sb)

        def wait2(buf, sa, sb):
            pltpu.make_async_copy(g_hbm.at[srcv.at[0, pl.ds(0, hc)]],
                                  buf.at[pl.ds(0, hc)], sa).wait()
            pltpu.make_async_copy(g_hbm.at[srcv.at[0, pl.ds(hc, hc)]],
                                  buf.at[pl.ds(hc, hc)], sb).wait()

        def seg_body(s, carry):
            off = pl.multiple_of(my_base + s * SEG, 8)
            pltpu.sync_copy(src_hbm.at[pl.ds(off, SEG)], srcv)
            pltpu.sync_copy(dst_hbm.at[pl.ds(off, SEG)], dstv)
            gather2(0, rows0, sem0, sem1)
            gather2(1, rows1, sem2, sem3)

            def step(t, c2):
                j = 2 * t
                wait2(rows0, sem0, sem1)
                pltpu.sync_copy(rows0, s_sh.at[dstv.at[j]], add=True)

                @pl.when(j + 2 < SEG)
                def _():
                    gather2(j + 2, rows0, sem0, sem1)

                wait2(rows1, sem2, sem3)
                pltpu.sync_copy(rows1, s_sh.at[dstv.at[j + 1]], add=True)

                @pl.when(j + 3 < SEG)
                def _():
                    gather2(j + 3, rows1, sem2, sem3)

                return c2

            lax.fori_loop(0, SEG // 2, step, 0)
            return carry

        lax.fori_loop(0, my_n // SEG, seg_body, 0)
        plsc.subcore_barrier()
        pltpu.sync_copy(s_sh.at[pl.ds(sid * rpt, rpt)],
                        s_out.at[cid, pl.ds(sid * rpt, rpt)])

    return pl.kernel(
        body,
        mesh=mesh,
        out_type=jax.ShapeDtypeStruct((NC, npad, h), jnp.float32),
        scratch_types=[
            pltpu.VMEM((SEG, EPC), jnp.int32),
            pltpu.VMEM((SEG, EPC), jnp.int32),
            pltpu.VMEM((EPC, h), jnp.float32),
            pltpu.VMEM((EPC, h), jnp.float32),
            pltpu.VMEM_SHARED((npad, h), jnp.float32),
            pltpu.SemaphoreType.DMA,
            pltpu.SemaphoreType.DMA,
            pltpu.SemaphoreType.DMA,
            pltpu.SemaphoreType.DMA,
        ],
        compiler_params=_sc_params(),
    )


BLK = 512


def _t1_body(x_ref, w_ref, da_ref, db_ref, o_ref):
    dinv = lax.rsqrt(da_ref[...] + db_ref[...] + 1.0)
    o_ref[...] = jnp.dot(x_ref[...], w_ref[...],
                         preferred_element_type=jnp.float32) * dinv


def _t2_body(n, sa_ref, sb_ref, g_ref, da_ref, db_ref, b_ref, w_ref, o_ref):
    i = pl.program_id(0)
    dinv = lax.rsqrt(da_ref[...] + db_ref[...] + 1.0)
    h1 = jnp.tanh((sa_ref[0] + sb_ref[0] + g_ref[...]) * dinv + b_ref[...])
    g2 = jnp.dot(h1, w_ref[...], preferred_element_type=jnp.float32) * dinv
    rows = lax.broadcasted_iota(jnp.int32, (BLK, 1), 0) + i * BLK
    o_ref[...] = jnp.where(rows < n, g2, 0.0)


def _t3_body(sa_ref, sb_ref, g_ref, da_ref, db_ref, b_ref, w_ref, bf_ref,
             o_ref):
    dinv = lax.rsqrt(da_ref[...] + db_ref[...] + 1.0)
    h2 = jnp.tanh((sa_ref[0] + sb_ref[0] + g_ref[...]) * dinv + b_ref[...])
    o_ref[...] = jnp.tanh(jnp.dot(h2, w_ref[...],
                                  preferred_element_type=jnp.float32)
                          + bf_ref[...])


def _row_spec(i_dim):
    return pl.BlockSpec((BLK, i_dim), lambda i: (i, 0))


def _part_spec(c):
    return pl.BlockSpec((1, BLK, H_BLK), lambda i, c=c: (c, i, 0))


H_BLK = 128


def _full_spec(a, b):
    return pl.BlockSpec((a, b), lambda i: (0, 0))


def kernel(x, edge_index, W1, b1, W2, b2, Wfc, bfc):
    n, d = x.shape
    h = W1.shape[1]
    e = edge_index.shape[1]
    npad, na, nb = _pad_sizes(n, e)
    pool = NS * (na + nb)
    epad = pool * EPC

    src = edge_index[0].astype(jnp.int32)
    dst = edge_index[1].astype(jnp.int32)
    pad_cycle = n + (jnp.arange(epad - e, dtype=jnp.int32) % (npad - n))
    pad_src = pad_cycle
    pad_dst = pad_cycle
    src_p = jnp.concatenate([src, pad_src]).reshape(pool, EPC)
    dst_p = jnp.concatenate([dst, pad_dst]).reshape(pool, EPC)
    x_p = jnp.pad(x, ((0, npad - n), (0, 0)))

    deg2 = _deg_call(npad, na, nb)(dst_p)
    dega = deg2[0].reshape(npad, 1)
    degb = deg2[1].reshape(npad, 1)

    grid = (npad // BLK,)

    g1 = pl.pallas_call(
        _t1_body,
        grid=grid,
        in_specs=[_row_spec(d), _full_spec(d, h), _row_spec(1), _row_spec(1)],
        out_specs=_row_spec(h),
        out_shape=jax.ShapeDtypeStruct((npad, h), jnp.float32),
    )(x_p, W1, dega, degb)

    scat = _scatter_call(npad, na, nb, h)
    s1 = scat(g1, src_p, dst_p)

    g2 = pl.pallas_call(
        functools.partial(_t2_body, n),
        grid=grid,
        in_specs=[_part_spec(0), _part_spec(1), _row_spec(h), _row_spec(1),
                  _row_spec(1), _full_spec(1, h), _full_spec(h, h)],
        out_specs=_row_spec(h),
        out_shape=jax.ShapeDtypeStruct((npad, h), jnp.float32),
    )(s1, s1, g1, dega, degb, b1.reshape(1, h), W2)

    s2 = scat(g2, src_p, dst_p)

    out = pl.pallas_call(
        _t3_body,
        grid=grid,
        in_specs=[_part_spec(0), _part_spec(1), _row_spec(h), _row_spec(1),
                  _row_spec(1), _full_spec(1, h), _full_spec(h, h),
                  _full_spec(1, h)],
        out_specs=_row_spec(h),
        out_shape=jax.ShapeDtypeStruct((npad, h), jnp.float32),
    )(s2, s2, g2, dega, degb, b2.reshape(1, h), Wfc,
      bfc.reshape(1, h))

    return out[:n]


# speedup vs baseline: 1.2533x; 1.0337x over previous
import functools

import numpy as np

import jax
import jax.numpy as jnp
from jax import lax
from jax.experimental import pallas as pl
from jax.experimental.pallas import tpu as pltpu
from jax.experimental.pallas import tpu_sc as plsc

NC = 2
NS = 16
EPC = 128
SEG = 40
SPLIT = (80, 80)
LANES = 128
_SC_PARAMS = None


def _sc_params():
    global _SC_PARAMS
    if _SC_PARAMS is None:
        _SC_PARAMS = pltpu.CompilerParams(needs_layout_passes=False)
    return _SC_PARAMS


def _pad_sizes(n, e):
    npad = ((n + 1 + LANES - 1) // LANES) * LANES
    npad = ((npad + NS * LANES - 1) // (NS * LANES)) * (NS * LANES)
    total = (e + EPC - 1) // EPC
    unit = NS * (SPLIT[0] + SPLIT[1])
    scale = (total + unit - 1) // unit
    na, nb = SPLIT[0] * scale, SPLIT[1] * scale
    return npad, na, nb


@functools.lru_cache(maxsize=None)
def _deg_call(npad, na, nb):
    nr = npad // LANES
    rpt = nr // NS
    pool = NS * (na + nb)
    mesh = plsc.VectorSubcoreMesh(core_axis_name="c", subcore_axis_name="s")

    def body(dst_hbm, deg_out, dstv, hist, rowidx, sdeg):
        cid = lax.axis_index("c")
        sid = lax.axis_index("s")
        my_n = jnp.where(cid == 0, na, nb)
        my_base = pl.multiple_of(
            jnp.where(cid == 0, sid * na, NS * na + sid * nb), 8)
        z16 = jnp.zeros((16,), jnp.float32)

        def zero_row(r, carry):
            for c in range(LANES // 16):
                hist[r, pl.ds(c * 16, 16)] = z16
            return carry

        lax.fori_loop(0, nr, zero_row, 0)
        for c in range(nr // 16):
            rowidx[pl.ds(c * 16, 16)] = lax.iota(jnp.int32, 16) + c * 16

        pltpu.sync_copy(hist.at[pl.ds(sid * rpt, rpt)],
                        sdeg.at[pl.ds(sid * rpt, rpt)])
        plsc.subcore_barrier()

        ones = jnp.ones((16,), jnp.float32)

        def seg_body(s, carry):
            off = pl.multiple_of(my_base + s * SEG, 8)
            pltpu.sync_copy(dst_hbm.at[pl.ds(off, SEG)], dstv)

            def acc(i, c2):
                r = i // (EPC // 16)
                c = i % (EPC // 16)
                idx = dstv[r, pl.ds(c * 16, 16)]
                row = jnp.right_shift(idx, 7)
                col = jnp.bitwise_and(idx, LANES - 1)
                plsc.addupdate_scatter(hist, [row, col], ones)
                return c2

            lax.fori_loop(0, SEG * (EPC // 16), acc, 0)
            return carry

        lax.fori_loop(0, my_n // SEG, seg_body, 0)

        pltpu.sync_copy(hist, sdeg.at[rowidx], add=True)
        plsc.subcore_barrier()

        @pl.when(sid == 0)
        def _():
            pltpu.sync_copy(sdeg, deg_out.at[cid])

    return pl.kernel(
        body,
        mesh=mesh,
        out_type=jax.ShapeDtypeStruct((NC, nr, LANES), jnp.float32),
        scratch_types=[
            pltpu.VMEM((SEG, EPC), jnp.int32),
            pltpu.VMEM((nr, LANES), jnp.float32),
            pltpu.VMEM((nr,), jnp.int32),
            pltpu.VMEM_SHARED((nr, LANES), jnp.float32),
        ],
        compiler_params=_sc_params(),
    )


@functools.lru_cache(maxsize=None)
def _scatter_call(npad, na, nb, h):
    rpt = npad // NS
    mesh = plsc.VectorSubcoreMesh(core_axis_name="c", subcore_axis_name="s")

    def body(g_hbm, src_hbm, dst_hbm, s_out,
             srcv, dstv, rows0, rows1, s_sh, sem0, sem1, sem2, sem3):
        cid = lax.axis_index("c")
        sid = lax.axis_index("s")
        my_n = jnp.where(cid == 0, na, nb)
        my_base = pl.multiple_of(
            jnp.where(cid == 0, sid * na, NS * na + sid * nb), 8)

        z16 = jnp.zeros((16,), jnp.float32)

        def zero_row(r, carry):
            for c in range(h // 16):
                rows0[r, pl.ds(c * 16, 16)] = z16
            return carry

        lax.fori_loop(0, EPC, zero_row, 0)

        def zs(k, carry):
            pltpu.sync_copy(rows0, s_sh.at[pl.ds(sid * rpt + k * EPC, EPC)])
            return carry

        lax.fori_loop(0, rpt // EPC, zs, 0)
        plsc.subcore_barrier()

        def seg_body(s, carry):
            off = pl.multiple_of(my_base + s * SEG, 8)
            pltpu.sync_copy(src_hbm.at[pl.ds(off, SEG)], srcv)
            pltpu.sync_copy(dst_hbm.at[pl.ds(off, SEG)], dstv)
            pltpu.async_copy(g_hbm.at[srcv.at[0]], rows0, sem0)
            pltpu.async_copy(g_hbm.at[srcv.at[1]], rows1, sem1)

            def step(t, c2):
                j = 2 * t
                pltpu.make_async_copy(g_hbm.at[srcv.at[0]], rows0, sem0).wait()
                pltpu.sync_copy(rows0, s_sh.at[dstv.at[j]], add=True)

                @pl.when(j + 2 < SEG)
                def _():
                    pltpu.async_copy(g_hbm.at[srcv.at[j + 2]], rows0, sem0)

                pltpu.make_async_copy(g_hbm.at[srcv.at[1]], rows1, sem1).wait()
                pltpu.sync_copy(rows1, s_sh.at[dstv.at[j + 1]], add=True)

                @pl.when(j + 3 < SEG)
                def _():
                    pltpu.async_copy(g_hbm.at[srcv.at[j + 3]], rows1, sem1)

                return c2

            lax.fori_loop(0, SEG // 2, step, 0)
            return carry

        lax.fori_loop(0, my_n // SEG, seg_body, 0)
        plsc.subcore_barrier()
        pltpu.sync_copy(s_sh.at[pl.ds(sid * rpt, rpt)],
                        s_out.at[cid, pl.ds(sid * rpt, rpt)])

    return pl.kernel(
        body,
        mesh=mesh,
        out_type=jax.ShapeDtypeStruct((NC, npad, h), jnp.float32),
        scratch_types=[
            pltpu.VMEM((SEG, EPC), jnp.int32),
            pltpu.VMEM((SEG, EPC), jnp.int32),
            pltpu.VMEM((EPC, h), jnp.float32),
            pltpu.VMEM((EPC, h), jnp.float32),
            pltpu.VMEM_SHARED((npad, h), jnp.float32),
            pltpu.SemaphoreType.DMA,
            pltpu.SemaphoreType.DMA,
            pltpu.SemaphoreType.DMA,
            pltpu.SemaphoreType.DMA,
        ],
        compiler_params=_sc_params(),
    )


BLK = 512


def _t1_body(x_ref, w_ref, da_ref, db_ref, o_ref):
    dinv = lax.rsqrt(da_ref[...] + db_ref[...] + 1.0)
    o_ref[...] = jnp.dot(x_ref[...], w_ref[...],
                         preferred_element_type=jnp.float32) * dinv


def _t2_body(n, sa_ref, sb_ref, g_ref, da_ref, db_ref, b_ref, w_ref, o_ref):
    i = pl.program_id(0)
    dinv = lax.rsqrt(da_ref[...] + db_ref[...] + 1.0)
    h1 = jnp.tanh((sa_ref[0] + sb_ref[0] + g_ref[...]) * dinv + b_ref[...])
    g2 = jnp.dot(h1, w_ref[...], preferred_element_type=jnp.float32) * dinv
    rows = lax.broadcasted_iota(jnp.int32, (BLK, 1), 0) + i * BLK
    o_ref[...] = jnp.where(rows < n, g2, 0.0)


def _t3_body(sa_ref, sb_ref, g_ref, da_ref, db_ref, b_ref, w_ref, bf_ref,
             o_ref):
    dinv = lax.rsqrt(da_ref[...] + db_ref[...] + 1.0)
    h2 = jnp.tanh((sa_ref[0] + sb_ref[0] + g_ref[...]) * dinv + b_ref[...])
    o_ref[...] = jnp.tanh(jnp.dot(h2, w_ref[...],
                                  preferred_element_type=jnp.float32)
                          + bf_ref[...])


def _row_spec(i_dim):
    return pl.BlockSpec((BLK, i_dim), lambda i: (i, 0))


def _part_spec(c):
    return pl.BlockSpec((1, BLK, H_BLK), lambda i, c=c: (c, i, 0))


H_BLK = 128


def _full_spec(a, b):
    return pl.BlockSpec((a, b), lambda i: (0, 0))


def kernel(x, edge_index, W1, b1, W2, b2, Wfc, bfc):
    n, d = x.shape
    h = W1.shape[1]
    e = edge_index.shape[1]
    npad, na, nb = _pad_sizes(n, e)
    pool = NS * (na + nb)
    epad = pool * EPC

    src = edge_index[0].astype(jnp.int32)
    dst = edge_index[1].astype(jnp.int32)
    pad_cycle = jnp.asarray(
        n + (np.arange(epad - e, dtype=np.int32) % (npad - n)))
    src_p = jnp.concatenate([src, pad_cycle]).reshape(pool, EPC)
    dst_p = jnp.concatenate([dst, pad_cycle]).reshape(pool, EPC)

    deg2 = _deg_call(npad, na, nb)(dst_p)
    dega = deg2[0].reshape(npad, 1)
    degb = deg2[1].reshape(npad, 1)

    grid = (npad // BLK,)

    g1 = pl.pallas_call(
        _t1_body,
        grid=grid,
        in_specs=[_row_spec(d), _full_spec(d, h), _row_spec(1), _row_spec(1)],
        out_specs=_row_spec(h),
        out_shape=jax.ShapeDtypeStruct((npad, h), jnp.float32),
    )(x, W1, dega, degb)

    scat = _scatter_call(npad, na, nb, h)
    s1 = scat(g1, src_p, dst_p)

    g2 = pl.pallas_call(
        functools.partial(_t2_body, n),
        grid=grid,
        in_specs=[_part_spec(0), _part_spec(1), _row_spec(h), _row_spec(1),
                  _row_spec(1), _full_spec(1, h), _full_spec(h, h)],
        out_specs=_row_spec(h),
        out_shape=jax.ShapeDtypeStruct((npad, h), jnp.float32),
    )(s1, s1, g1, dega, degb, b1.reshape(1, h), W2)

    s2 = scat(g2, src_p, dst_p)

    out = pl.pallas_call(
        _t3_body,
        grid=grid,
        in_specs=[_part_spec(0), _part_spec(1), _row_spec(h), _row_spec(1),
                  _row_spec(1), _full_spec(1, h), _full_spec(h, h),
                  _full_spec(1, h)],
        out_specs=_row_spec(h),
        out_shape=jax.ShapeDtypeStruct((n, h), jnp.float32),
    )(s2, s2, g2, dega, degb, b2.reshape(1, h), Wfc,
      bfc.reshape(1, h))

    return out


# speedup vs baseline: 1.2937x; 1.0322x over previous
import functools

import numpy as np

import jax
import jax.numpy as jnp
from jax import lax
from jax.experimental import pallas as pl
from jax.experimental.pallas import tpu as pltpu
from jax.experimental.pallas import tpu_sc as plsc

NC = 2
NS = 16
EPC = 128
SEG = 40
SPLIT = (80, 80)
LANES = 128
_SC_PARAMS = None


def _sc_params():
    global _SC_PARAMS
    if _SC_PARAMS is None:
        _SC_PARAMS = pltpu.CompilerParams(needs_layout_passes=False)
    return _SC_PARAMS


def _pad_sizes(n, e):
    npad = ((n + 1 + LANES - 1) // LANES) * LANES
    npad = ((npad + NS * LANES - 1) // (NS * LANES)) * (NS * LANES)
    total = (e + EPC - 1) // EPC
    unit = NS * (SPLIT[0] + SPLIT[1])
    scale = (total + unit - 1) // unit
    na, nb = SPLIT[0] * scale, SPLIT[1] * scale
    return npad, na, nb


@functools.lru_cache(maxsize=None)
def _deg_call(npad, na, nb):
    nr = npad // LANES
    rpt = nr // NS
    pool = NS * (na + nb)
    mesh = plsc.VectorSubcoreMesh(core_axis_name="c", subcore_axis_name="s")

    def body(edge_hbm, deg_out, dstv, hist, rowidx, sdeg):
        cid = lax.axis_index("c")
        sid = lax.axis_index("s")
        my_n = jnp.where(cid == 0, na, nb)
        my_base = pl.multiple_of(
            jnp.where(cid == 0, sid * na, NS * na + sid * nb), 8)
        z16 = jnp.zeros((16,), jnp.float32)

        def zero_row(r, carry):
            for c in range(LANES // 16):
                hist[r, pl.ds(c * 16, 16)] = z16
            return carry

        lax.fori_loop(0, nr, zero_row, 0)
        for c in range(nr // 16):
            rowidx[pl.ds(c * 16, 16)] = lax.iota(jnp.int32, 16) + c * 16

        pltpu.sync_copy(hist.at[pl.ds(sid * rpt, rpt)],
                        sdeg.at[pl.ds(sid * rpt, rpt)])
        plsc.subcore_barrier()

        ones = jnp.ones((16,), jnp.float32)

        def seg_body(s, carry):
            off = pl.multiple_of(my_base + s * SEG, 8)
            pltpu.sync_copy(edge_hbm.at[1, pl.ds(off, SEG)], dstv)

            def acc(i, c2):
                r = i // (EPC // 16)
                c = i % (EPC // 16)
                idx = dstv[r, pl.ds(c * 16, 16)]
                row = jnp.right_shift(idx, 7)
                col = jnp.bitwise_and(idx, LANES - 1)
                plsc.addupdate_scatter(hist, [row, col], ones)
                return c2

            lax.fori_loop(0, SEG * (EPC // 16), acc, 0)
            return carry

        lax.fori_loop(0, my_n // SEG, seg_body, 0)

        pltpu.sync_copy(hist, sdeg.at[rowidx], add=True)
        plsc.subcore_barrier()

        @pl.when(sid == 0)
        def _():
            pltpu.sync_copy(sdeg, deg_out.at[cid])

    return pl.kernel(
        body,
        mesh=mesh,
        out_type=jax.ShapeDtypeStruct((NC, nr, LANES), jnp.float32),
        scratch_types=[
            pltpu.VMEM((SEG, EPC), jnp.int32),
            pltpu.VMEM((nr, LANES), jnp.float32),
            pltpu.VMEM((nr,), jnp.int32),
            pltpu.VMEM_SHARED((nr, LANES), jnp.float32),
        ],
        compiler_params=_sc_params(),
    )


@functools.lru_cache(maxsize=None)
def _scatter_call(npad, na, nb, h):
    rpt = npad // NS
    mesh = plsc.VectorSubcoreMesh(core_axis_name="c", subcore_axis_name="s")

    def body(g_hbm, edge_hbm, s_out,
             srcv, dstv, rows0, rows1, s_sh, sem0, sem1):
        cid = lax.axis_index("c")
        sid = lax.axis_index("s")
        my_n = jnp.where(cid == 0, na, nb)
        my_base = pl.multiple_of(
            jnp.where(cid == 0, sid * na, NS * na + sid * nb), 8)

        z16 = jnp.zeros((16,), jnp.float32)

        def zero_row(r, carry):
            for c in range(h // 16):
                rows0[r, pl.ds(c * 16, 16)] = z16
            return carry

        lax.fori_loop(0, EPC, zero_row, 0)

        def zs(k, carry):
            pltpu.sync_copy(rows0, s_sh.at[pl.ds(sid * rpt + k * EPC, EPC)])
            return carry

        lax.fori_loop(0, rpt // EPC, zs, 0)
        plsc.subcore_barrier()

        def seg_body(s, carry):
            off = pl.multiple_of(my_base + s * SEG, 8)
            pltpu.sync_copy(edge_hbm.at[0, pl.ds(off, SEG)], srcv)
            pltpu.sync_copy(edge_hbm.at[1, pl.ds(off, SEG)], dstv)
            pltpu.async_copy(g_hbm.at[srcv.at[0]], rows0, sem0)
            pltpu.async_copy(g_hbm.at[srcv.at[1]], rows1, sem1)

            def step(t, c2):
                j = 2 * t
                pltpu.make_async_copy(g_hbm.at[srcv.at[0]], rows0, sem0).wait()
                pltpu.sync_copy(rows0, s_sh.at[dstv.at[j]], add=True)

                @pl.when(j + 2 < SEG)
                def _():
                    pltpu.async_copy(g_hbm.at[srcv.at[j + 2]], rows0, sem0)

                pltpu.make_async_copy(g_hbm.at[srcv.at[1]], rows1, sem1).wait()
                pltpu.sync_copy(rows1, s_sh.at[dstv.at[j + 1]], add=True)

                @pl.when(j + 3 < SEG)
                def _():
                    pltpu.async_copy(g_hbm.at[srcv.at[j + 3]], rows1, sem1)

                return c2

            lax.fori_loop(0, SEG // 2, step, 0)
            return carry

        lax.fori_loop(0, my_n // SEG, seg_body, 0)
        plsc.subcore_barrier()
        pltpu.sync_copy(s_sh.at[pl.ds(sid * rpt, rpt)],
                        s_out.at[cid, pl.ds(sid * rpt, rpt)])

    return pl.kernel(
        body,
        mesh=mesh,
        out_type=jax.ShapeDtypeStruct((NC, npad, h), jnp.float32),
        scratch_types=[
            pltpu.VMEM((SEG, EPC), jnp.int32),
            pltpu.VMEM((SEG, EPC), jnp.int32),
            pltpu.VMEM((EPC, h), jnp.float32),
            pltpu.VMEM((EPC, h), jnp.float32),
            pltpu.VMEM_SHARED((npad, h), jnp.float32),
            pltpu.SemaphoreType.DMA,
            pltpu.SemaphoreType.DMA,
        ],
        compiler_params=_sc_params(),
    )


BLK = 512


def _t1_body(x_ref, w_ref, da_ref, db_ref, o_ref):
    dinv = lax.rsqrt(da_ref[...] + db_ref[...] + 1.0)
    o_ref[...] = jnp.dot(x_ref[...], w_ref[...],
                         preferred_element_type=jnp.float32) * dinv


def _t2_body(n, sa_ref, sb_ref, g_ref, da_ref, db_ref, b_ref, w_ref, o_ref):
    i = pl.program_id(0)
    dinv = lax.rsqrt(da_ref[...] + db_ref[...] + 1.0)
    h1 = jnp.tanh((sa_ref[0] + sb_ref[0] + g_ref[...]) * dinv + b_ref[...])
    g2 = jnp.dot(h1, w_ref[...], preferred_element_type=jnp.float32) * dinv
    rows = lax.broadcasted_iota(jnp.int32, (BLK, 1), 0) + i * BLK
    o_ref[...] = jnp.where(rows < n, g2, 0.0)


def _t3_body(sa_ref, sb_ref, g_ref, da_ref, db_ref, b_ref, w_ref, bf_ref,
             o_ref):
    dinv = lax.rsqrt(da_ref[...] + db_ref[...] + 1.0)
    h2 = jnp.tanh((sa_ref[0] + sb_ref[0] + g_ref[...]) * dinv + b_ref[...])
    o_ref[...] = jnp.tanh(jnp.dot(h2, w_ref[...],
                                  preferred_element_type=jnp.float32)
                          + bf_ref[...])


def _row_spec(i_dim):
    return pl.BlockSpec((BLK, i_dim), lambda i: (i, 0))


def _part_spec(c):
    return pl.BlockSpec((1, BLK, H_BLK), lambda i, c=c: (c, i, 0))


H_BLK = 128


def _full_spec(a, b):
    return pl.BlockSpec((a, b), lambda i: (0, 0))


def kernel(x, edge_index, W1, b1, W2, b2, Wfc, bfc):
    n, d = x.shape
    h = W1.shape[1]
    e = edge_index.shape[1]
    npad, na, nb = _pad_sizes(n, e)
    pool = NS * (na + nb)
    epad = pool * EPC

    pad_np = (n + (np.arange(epad - e, dtype=np.int32) % (npad - n)))
    pad2 = jnp.asarray(
        np.broadcast_to(pad_np.reshape(1, -1, EPC), (2, (epad - e) // EPC,
                                                     EPC)))
    edge_p = jnp.concatenate(
        [edge_index.astype(jnp.int32).reshape(2, e // EPC, EPC), pad2],
        axis=1)

    deg2 = _deg_call(npad, na, nb)(edge_p)
    dega = deg2[0].reshape(npad, 1)
    degb = deg2[1].reshape(npad, 1)

    grid = (npad // BLK,)

    g1 = pl.pallas_call(
        _t1_body,
        grid=grid,
        in_specs=[_row_spec(d), _full_spec(d, h), _row_spec(1), _row_spec(1)],
        out_specs=_row_spec(h),
        out_shape=jax.ShapeDtypeStruct((npad, h), jnp.float32),
    )(x, W1, dega, degb)

    scat = _scatter_call(npad, na, nb, h)
    s1 = scat(g1, edge_p)

    g2 = pl.pallas_call(
        functools.partial(_t2_body, n),
        grid=grid,
        in_specs=[_part_spec(0), _part_spec(1), _row_spec(h), _row_spec(1),
                  _row_spec(1), _full_spec(1, h), _full_spec(h, h)],
        out_specs=_row_spec(h),
        out_shape=jax.ShapeDtypeStruct((npad, h), jnp.float32),
    )(s1, s1, g1, dega, degb, b1.reshape(1, h), W2)

    s2 = scat(g2, edge_p)

    out = pl.pallas_call(
        _t3_body,
        grid=grid,
        in_specs=[_part_spec(0), _part_spec(1), _row_spec(h), _row_spec(1),
                  _row_spec(1), _full_spec(1, h), _full_spec(h, h),
                  _full_spec(1, h)],
        out_specs=_row_spec(h),
        out_shape=jax.ShapeDtypeStruct((n, h), jnp.float32),
    )(s2, s2, g2, dega, degb, b2.reshape(1, h), Wfc,
      bfc.reshape(1, h))

    return out
